# baseline (speedup 1.0000x reference)
import functools

import jax
import jax.numpy as jnp
from jax import lax
from jax.experimental import pallas as pl
from jax.experimental.pallas import tpu as pltpu
from jax.experimental.pallas import tpu_sc as plsc

_NUM_CORES = 2
_NUM_SUBCORES = 16
_NUM_WORKERS = _NUM_CORES * _NUM_SUBCORES


def _sc_copy_body(pos_hbm, out_hbm, sem):
    rows = pos_hbm.shape[0] // _NUM_WORKERS
    wid = lax.axis_index("s") * _NUM_CORES + lax.axis_index("c")
    base = wid * rows
    pltpu.async_copy(
        pos_hbm.at[pl.ds(base, rows)],
        out_hbm.at[pl.ds(base, rows)],
        sem,
    ).wait()


def kernel(x, pos_emb):
    seq_len = x.shape[1]
    d = pos_emb.shape[1]
    table = pos_emb[:seq_len]
    mesh = plsc.VectorSubcoreMesh(core_axis_name="c", subcore_axis_name="s")
    out = pl.kernel(
        _sc_copy_body,
        mesh=mesh,
        out_type=jax.ShapeDtypeStruct((seq_len, d), pos_emb.dtype),
        scratch_types=[pltpu.SemaphoreType.DMA],
    )(table)
    return out[None]


# speedup vs baseline: 8.8328x; 8.8328x over previous
import functools

import jax
import jax.numpy as jnp
from jax import lax
from jax.experimental import pallas as pl
from jax.experimental.pallas import tpu as pltpu
from jax.experimental.pallas import tpu_sc as plsc

_NUM_CORES = 2
_NUM_SUBCORES = 16
_NUM_WORKERS = _NUM_CORES * _NUM_SUBCORES


def _sc_copy_body(pos_hbm, out_hbm, buf, sem):
    rows = pos_hbm.shape[0] // _NUM_WORKERS
    wid = lax.axis_index("s") * _NUM_CORES + lax.axis_index("c")
    base = wid * rows
    pltpu.async_copy(pos_hbm.at[pl.ds(base, rows)], buf, sem).wait()
    pltpu.async_copy(buf, out_hbm.at[pl.ds(base, rows)], sem).wait()


def kernel(x, pos_emb):
    seq_len = x.shape[1]
    d = pos_emb.shape[1]
    table = pos_emb[:seq_len]
    mesh = plsc.VectorSubcoreMesh(core_axis_name="c", subcore_axis_name="s")
    out = pl.kernel(
        _sc_copy_body,
        mesh=mesh,
        out_type=jax.ShapeDtypeStruct((seq_len, d), pos_emb.dtype),
        scratch_types=[
            pltpu.VMEM((seq_len // _NUM_WORKERS, d), pos_emb.dtype),
            pltpu.SemaphoreType.DMA,
        ],
    )(table)
    return out[None]
